# baseline (speedup 1.0000x reference)
import functools

import jax
import jax.numpy as jnp
from jax import lax
from jax.experimental import pallas as pl
from jax.experimental.pallas import tpu as pltpu
from jax.experimental.pallas import tpu_sc as plsc

_NUM_CODES = 8192
_CODE_DIM = 256
_N_TOKENS = 16384
_BM = 512
_SCALE = 1.25 / (_N_TOKENS * _CODE_DIM)
_GW = 128
_N_GROUPS = _NUM_CODES // _GW
_RS = 64
_CSQ_BOUND = 2.0 ** -18


def _dist_body(z_ref, cb_ref, csq_ref, idx_ref, loss_ref, acc_ref):
    i = pl.program_id(0)
    z = z_ref[...]
    cb = cb_ref[...]
    csq = csq_ref[...]
    zsq = jnp.sum(z * z, axis=1, keepdims=True)
    mm2 = lax.dot_general(z + z, cb, (((1,), (1,)), ((), ())),
                          preferred_element_type=jnp.float32)

    fast_ok = jnp.logical_and(jnp.min(zsq) >= 129.0,
                              jnp.max(csq) < _CSQ_BOUND)

    def sweep(fast):
        lane = lax.broadcasted_iota(jnp.int32, (_RS, _GW), 1)
        part = None
        for r in range(0, _BM, _RS):
            zsq_r = zsq[r:r + _RS]

            def dist_g(g):
                m = mm2[r:r + _RS, g * _GW:(g + 1) * _GW]
                if fast:
                    return zsq_r - m
                return (zsq_r + csq[:, g * _GW:(g + 1) * _GW]) - m

            rmin = dist_g(0)
            rgrp = jnp.zeros((_RS, _GW), jnp.int32)
            for g in range(1, _N_GROUPS):
                dg = dist_g(g)
                lt = dg < rmin
                rmin = jnp.where(lt, dg, rmin)
                rgrp = jnp.where(lt, g, rgrp)

            cidx = rgrp * _GW + lane
            dmin = jnp.min(rmin, axis=1, keepdims=True)
            cand = jnp.where(rmin == dmin, cidx, _NUM_CODES)
            idx_ref[r:r + _RS, :] = jnp.min(cand, axis=1, keepdims=True)
            ps = jnp.sum(dmin)
            part = ps if part is None else part + ps
        acc_ref[0] += part

    @pl.when(i == 0)
    def _():
        acc_ref[0] = 0.0

    @pl.when(fast_ok)
    def _():
        sweep(True)

    @pl.when(jnp.logical_not(fast_ok))
    def _():
        sweep(False)

    loss_ref[0] = acc_ref[0] * _SCALE


def _dist_call(z, codebook, csq_row):
    steps = _N_TOKENS // _BM
    return pl.pallas_call(
        _dist_body,
        grid=(steps,),
        in_specs=[
            pl.BlockSpec((_BM, _CODE_DIM), lambda i: (i, 0)),
            pl.BlockSpec((_NUM_CODES, _CODE_DIM), lambda i: (0, 0)),
            pl.BlockSpec((1, _NUM_CODES), lambda i: (0, 0)),
        ],
        out_specs=[
            pl.BlockSpec((_BM, 1), lambda i: (i, 0)),
            pl.BlockSpec(memory_space=pltpu.SMEM),
        ],
        out_shape=[
            jax.ShapeDtypeStruct((_N_TOKENS, 1), jnp.int32),
            jax.ShapeDtypeStruct((1,), jnp.float32),
        ],
        scratch_shapes=[pltpu.SMEM((1,), jnp.float32)],
    )(z, codebook, csq_row)


_N_WORKERS = 32
_B_PER_W = _N_TOKENS // _N_WORKERS
_CHUNK = 128


def _gather_body(idx_hbm, cb_hbm, out_hbm, idx_v, rows_v, sem):
    wid = lax.axis_index("s") * 2 + lax.axis_index("c")
    for c in range(_B_PER_W // _CHUNK):
        base = wid * _B_PER_W + c * _CHUNK
        pltpu.sync_copy(idx_hbm.at[pl.ds(base, _CHUNK)], idx_v)
        pltpu.async_copy(cb_hbm.at[idx_v], rows_v, sem).wait()
        pltpu.sync_copy(rows_v, out_hbm.at[pl.ds(base, _CHUNK)])


def _gather_rows(indices, codebook):
    mesh = plsc.VectorSubcoreMesh(core_axis_name="c", subcore_axis_name="s")
    gk = functools.partial(
        pl.kernel,
        mesh=mesh,
        out_type=jax.ShapeDtypeStruct((_N_TOKENS, _CODE_DIM), jnp.float32),
        scratch_types=[
            pltpu.VMEM((_CHUNK,), jnp.int32),
            pltpu.VMEM((_CHUNK, _CODE_DIM), jnp.float32),
            pltpu.SemaphoreType.DMA,
        ],
    )(_gather_body)
    return gk(indices, codebook)


def kernel(z, codebook):
    csq_row = jnp.sum(codebook * codebook, axis=1).reshape(1, _NUM_CODES)
    idx, loss = _dist_call(z, codebook, csq_row)
    indices = idx.reshape(_N_TOKENS)
    z_q = _gather_rows(indices, codebook)
    return (z_q, indices, loss[0])


# speedup vs baseline: 1.1993x; 1.1993x over previous
import functools

import jax
import jax.numpy as jnp
from jax import lax
from jax.experimental import pallas as pl
from jax.experimental.pallas import tpu as pltpu
from jax.experimental.pallas import tpu_sc as plsc

_NUM_CODES = 8192
_CODE_DIM = 256
_N_TOKENS = 16384
_BM = 512
_SCALE = 1.25 / (_N_TOKENS * _CODE_DIM)
_GW = 128
_N_GROUPS = _NUM_CODES // _GW
_N_QUADS = _N_GROUPS // 4
_RS = 64
_CSQ_BOUND = 2.0 ** -18


def _make_dist_body(fast):
    def body(z_ref, cb_ref, csq_ref, zsq_ref, idx_ref, loss_ref, acc_ref):
        i = pl.program_id(0)
        z = z_ref[...]
        cb = cb_ref[...]
        csq = csq_ref[...]
        mm2 = lax.dot_general(z + z, cb, (((1,), (1,)), ((), ())),
                              preferred_element_type=jnp.float32)

        lane = lax.broadcasted_iota(jnp.int32, (_RS, _GW), 1)
        part = None
        for r in range(0, _BM, _RS):
            zsq_r = zsq_ref[r:r + _RS]

            def dist_g(g):
                m = mm2[r:r + _RS, g * _GW:(g + 1) * _GW]
                if fast:
                    return zsq_r - m
                return (zsq_r + csq[:, g * _GW:(g + 1) * _GW]) - m

            def quad(q):
                d0 = dist_g(4 * q)
                d1 = dist_g(4 * q + 1)
                d2 = dist_g(4 * q + 2)
                d3 = dist_g(4 * q + 3)
                m01 = jnp.minimum(d0, d1)
                m23 = jnp.minimum(d2, d3)
                return jnp.minimum(m01, m23), m01, d0, d2

            rmin, rm01, rd0, rd2 = quad(0)
            rq = jnp.zeros((_RS, _GW), jnp.int32)
            for q in range(1, _N_QUADS):
                mq, m01, d0, d2 = quad(q)
                lt = mq < rmin
                rmin = jnp.where(lt, mq, rmin)
                rq = jnp.where(lt, q, rq)
                rm01 = jnp.where(lt, m01, rm01)
                rd0 = jnp.where(lt, d0, rd0)
                rd2 = jnp.where(lt, d2, rd2)

            s01 = jnp.where(rd0 == rm01, 0, 1)
            s23 = jnp.where(rd2 == rmin, 2, 3)
            s = jnp.where(rm01 == rmin, s01, s23)
            cidx = (rq * 4 + s) * _GW + lane

            dmin = jnp.min(rmin, axis=1, keepdims=True)
            cand = jnp.where(rmin == dmin, cidx, _NUM_CODES)
            idx_ref[r:r + _RS, :] = jnp.min(cand, axis=1, keepdims=True)
            ps = jnp.sum(dmin)
            part = ps if part is None else part + ps

        @pl.when(i == 0)
        def _():
            acc_ref[0] = 0.0

        acc_ref[0] += part
        loss_ref[0] = acc_ref[0] * _SCALE

    return body


def _dist_call(z, codebook, csq_row, zsq, fast):
    steps = _N_TOKENS // _BM
    return pl.pallas_call(
        _make_dist_body(fast),
        grid=(steps,),
        in_specs=[
            pl.BlockSpec((_BM, _CODE_DIM), lambda i: (i, 0)),
            pl.BlockSpec((_NUM_CODES, _CODE_DIM), lambda i: (0, 0)),
            pl.BlockSpec((1, _NUM_CODES), lambda i: (0, 0)),
            pl.BlockSpec((_BM, 1), lambda i: (i, 0)),
        ],
        out_specs=[
            pl.BlockSpec((_BM, 1), lambda i: (i, 0)),
            pl.BlockSpec(memory_space=pltpu.SMEM),
        ],
        out_shape=[
            jax.ShapeDtypeStruct((_N_TOKENS, 1), jnp.int32),
            jax.ShapeDtypeStruct((1,), jnp.float32),
        ],
        scratch_shapes=[pltpu.SMEM((1,), jnp.float32)],
    )(z, codebook, csq_row, zsq)


_N_WORKERS = 32
_B_PER_W = _N_TOKENS // _N_WORKERS
_CHUNK = 128


def _gather_body(idx_hbm, cb_hbm, out_hbm, idx_v, rows_v, sem):
    wid = lax.axis_index("s") * 2 + lax.axis_index("c")
    for c in range(_B_PER_W // _CHUNK):
        base = wid * _B_PER_W + c * _CHUNK
        pltpu.sync_copy(idx_hbm.at[pl.ds(base, _CHUNK)], idx_v)
        pltpu.async_copy(cb_hbm.at[idx_v], rows_v, sem).wait()
        pltpu.sync_copy(rows_v, out_hbm.at[pl.ds(base, _CHUNK)])


def _gather_rows(indices, codebook):
    mesh = plsc.VectorSubcoreMesh(core_axis_name="c", subcore_axis_name="s")
    gk = functools.partial(
        pl.kernel,
        mesh=mesh,
        out_type=jax.ShapeDtypeStruct((_N_TOKENS, _CODE_DIM), jnp.float32),
        scratch_types=[
            pltpu.VMEM((_CHUNK,), jnp.int32),
            pltpu.VMEM((_CHUNK, _CODE_DIM), jnp.float32),
            pltpu.SemaphoreType.DMA,
        ],
    )(_gather_body)
    return gk(indices, codebook)


def _run(z, codebook, csq_row, zsq, fast):
    idx, loss = _dist_call(z, codebook, csq_row, zsq, fast)
    indices = idx.reshape(_N_TOKENS)
    z_q = _gather_rows(indices, codebook)
    return (z_q, indices, loss[0])


def kernel(z, codebook):
    csq_col = jnp.sum(codebook * codebook, axis=1, keepdims=True)
    csq_row = csq_col.reshape(1, _NUM_CODES)
    zsq = jnp.sum(z * z, axis=1, keepdims=True)
    fast_ok = jnp.logical_and(jnp.min(zsq) >= 129.0,
                              jnp.max(csq_col) < _CSQ_BOUND)
    return lax.cond(
        fast_ok,
        lambda: _run(z, codebook, csq_row, zsq, True),
        lambda: _run(z, codebook, csq_row, zsq, False),
    )


# speedup vs baseline: 1.4324x; 1.1944x over previous
import functools

import jax
import jax.numpy as jnp
from jax import lax
from jax.experimental import pallas as pl
from jax.experimental.pallas import tpu as pltpu
from jax.experimental.pallas import tpu_sc as plsc

_NUM_CODES = 8192
_CODE_DIM = 256
_N_TOKENS = 16384
_BM = 512
_SCALE = 1.25 / (_N_TOKENS * _CODE_DIM)
_GW = 128
_N_GROUPS = _NUM_CODES // _GW
_RS = 64
_CSQ_BOUND = 2.0 ** -18
_CSQ_BOUND_SAFE = _CSQ_BOUND * (1.0 - 2.0 ** -10)


def _sweep(mm2, zsq, csq, idx_ref, fast):
    lane = lax.broadcasted_iota(jnp.int32, (_RS, _GW), 1)
    part = None
    for r in range(0, _BM, _RS):
        zsq_r = zsq[r:r + _RS]

        def dist_g(g):
            m = mm2[r:r + _RS, g * _GW:(g + 1) * _GW]
            if fast:
                return zsq_r - m
            return (zsq_r + csq[:, g * _GW:(g + 1) * _GW]) - m

        rmin = dist_g(0)
        rgrp = jnp.zeros((_RS, _GW), jnp.int32)
        for g in range(1, _N_GROUPS):
            dg = dist_g(g)
            lt = dg < rmin
            rmin = jnp.where(lt, dg, rmin)
            rgrp = jnp.where(lt, g, rgrp)

        cidx = rgrp * _GW + lane
        dmin = jnp.min(rmin, axis=1, keepdims=True)
        cand = jnp.where(rmin == dmin, cidx, _NUM_CODES)
        idx_ref[r:r + _RS, :] = jnp.min(cand, axis=1, keepdims=True)
        ps = jnp.sum(dmin)
        part = ps if part is None else part + ps
    return part


def _fast_body(z_ref, cb_ref, idx_ref, loss_ref, ok_ref, acc_ref, okacc_ref):
    i = pl.program_id(0)
    z = z_ref[...]
    cb = cb_ref[...]
    zsq = jnp.sum(z * z, axis=1, keepdims=True)
    mm2 = lax.dot_general(z + z, cb, (((1,), (1,)), ((), ())),
                          preferred_element_type=jnp.float32)

    stripe = _NUM_CODES // (_N_TOKENS // _BM)
    cs = cb_ref[pl.ds(i * stripe, stripe), :]
    csq_part = jnp.max(jnp.sum(cs * cs, axis=1))
    blk_ok = jnp.logical_and(jnp.min(zsq) >= 129.0,
                             csq_part < _CSQ_BOUND_SAFE).astype(jnp.int32)

    @pl.when(i == 0)
    def _():
        acc_ref[0] = 0.0
        okacc_ref[0] = 1

    okacc_ref[0] = jnp.minimum(okacc_ref[0], blk_ok)
    acc_ref[0] += _sweep(mm2, zsq, None, idx_ref, True)
    loss_ref[0] = acc_ref[0] * _SCALE
    ok_ref[0] = okacc_ref[0]


def _exact_body(z_ref, cb_ref, csq_ref, zsq_ref, idx_ref, loss_ref, acc_ref):
    i = pl.program_id(0)
    z = z_ref[...]
    cb = cb_ref[...]
    csq = csq_ref[...]
    mm2 = lax.dot_general(z + z, cb, (((1,), (1,)), ((), ())),
                          preferred_element_type=jnp.float32)

    @pl.when(i == 0)
    def _():
        acc_ref[0] = 0.0

    acc_ref[0] += _sweep(mm2, zsq_ref[...], csq, idx_ref, False)
    loss_ref[0] = acc_ref[0] * _SCALE


def _fast_call(z, codebook):
    steps = _N_TOKENS // _BM
    return pl.pallas_call(
        _fast_body,
        grid=(steps,),
        in_specs=[
            pl.BlockSpec((_BM, _CODE_DIM), lambda i: (i, 0)),
            pl.BlockSpec((_NUM_CODES, _CODE_DIM), lambda i: (0, 0)),
        ],
        out_specs=[
            pl.BlockSpec((_BM, 1), lambda i: (i, 0)),
            pl.BlockSpec(memory_space=pltpu.SMEM),
            pl.BlockSpec(memory_space=pltpu.SMEM),
        ],
        out_shape=[
            jax.ShapeDtypeStruct((_N_TOKENS, 1), jnp.int32),
            jax.ShapeDtypeStruct((1,), jnp.float32),
            jax.ShapeDtypeStruct((1,), jnp.int32),
        ],
        scratch_shapes=[pltpu.SMEM((1,), jnp.float32),
                        pltpu.SMEM((1,), jnp.int32)],
    )(z, codebook)


def _exact_call(z, codebook, csq_row, zsq):
    steps = _N_TOKENS // _BM
    return pl.pallas_call(
        _exact_body,
        grid=(steps,),
        in_specs=[
            pl.BlockSpec((_BM, _CODE_DIM), lambda i: (i, 0)),
            pl.BlockSpec((_NUM_CODES, _CODE_DIM), lambda i: (0, 0)),
            pl.BlockSpec((1, _NUM_CODES), lambda i: (0, 0)),
            pl.BlockSpec((_BM, 1), lambda i: (i, 0)),
        ],
        out_specs=[
            pl.BlockSpec((_BM, 1), lambda i: (i, 0)),
            pl.BlockSpec(memory_space=pltpu.SMEM),
        ],
        out_shape=[
            jax.ShapeDtypeStruct((_N_TOKENS, 1), jnp.int32),
            jax.ShapeDtypeStruct((1,), jnp.float32),
        ],
        scratch_shapes=[pltpu.SMEM((1,), jnp.float32)],
    )(z, codebook, csq_row, zsq)


_N_WORKERS = 32
_B_PER_W = _N_TOKENS // _N_WORKERS
_CHUNK = 128


def _gather_body(idx_hbm, cb_hbm, out_hbm, idx_v, rows_v, sem):
    wid = lax.axis_index("s") * 2 + lax.axis_index("c")
    for c in range(_B_PER_W // _CHUNK):
        base = wid * _B_PER_W + c * _CHUNK
        pltpu.sync_copy(idx_hbm.at[pl.ds(base, _CHUNK)], idx_v)
        pltpu.async_copy(cb_hbm.at[idx_v], rows_v, sem).wait()
        pltpu.sync_copy(rows_v, out_hbm.at[pl.ds(base, _CHUNK)])


def _gather_rows(indices, codebook):
    mesh = plsc.VectorSubcoreMesh(core_axis_name="c", subcore_axis_name="s")
    gk = functools.partial(
        pl.kernel,
        mesh=mesh,
        out_type=jax.ShapeDtypeStruct((_N_TOKENS, _CODE_DIM), jnp.float32),
        scratch_types=[
            pltpu.VMEM((_CHUNK,), jnp.int32),
            pltpu.VMEM((_CHUNK, _CODE_DIM), jnp.float32),
            pltpu.SemaphoreType.DMA,
        ],
    )(_gather_body)
    return gk(indices, codebook)


def kernel(z, codebook):
    idx_f, loss_f, ok = _fast_call(z, codebook)

    def use_fast():
        return idx_f, loss_f[0]

    def do_exact():
        csq_row = jnp.sum(codebook * codebook, axis=1).reshape(1, _NUM_CODES)
        zsq = jnp.sum(z * z, axis=1, keepdims=True)
        idx_e, loss_e = _exact_call(z, codebook, csq_row, zsq)
        return idx_e, loss_e[0]

    idx, loss = lax.cond(ok[0] > 0, use_fast, do_exact)
    indices = idx.reshape(_N_TOKENS)
    z_q = _gather_rows(indices, codebook)
    return (z_q, indices, loss)


# speedup vs baseline: 1.4862x; 1.0376x over previous
import functools

import jax
import jax.numpy as jnp
from jax import lax
from jax.experimental import pallas as pl
from jax.experimental.pallas import tpu as pltpu
from jax.experimental.pallas import tpu_sc as plsc

_NUM_CODES = 8192
_CODE_DIM = 256
_N_TOKENS = 16384
_BM = 1024
_SCALE = 1.25 / (_N_TOKENS * _CODE_DIM)
_GW = 128
_N_GROUPS = _NUM_CODES // _GW
_RS = 64
_CSQ_BOUND = 2.0 ** -18
_CSQ_BOUND_SAFE = _CSQ_BOUND * (1.0 - 2.0 ** -10)


def _sweep(mm2, zsq, csq, idx_ref, fast):
    lane = lax.broadcasted_iota(jnp.int32, (_RS, _GW), 1)
    part = None
    for r in range(0, _BM, _RS):
        zsq_r = zsq[r:r + _RS]

        def dist_g(g):
            m = mm2[r:r + _RS, g * _GW:(g + 1) * _GW]
            if fast:
                return zsq_r - m
            return (zsq_r + csq[:, g * _GW:(g + 1) * _GW]) - m

        rmin = dist_g(0)
        rgrp = jnp.zeros((_RS, _GW), jnp.int32)
        for g in range(1, _N_GROUPS):
            dg = dist_g(g)
            lt = dg < rmin
            rmin = jnp.where(lt, dg, rmin)
            rgrp = jnp.where(lt, g, rgrp)

        cidx = rgrp * _GW + lane
        dmin = jnp.min(rmin, axis=1, keepdims=True)
        cand = jnp.where(rmin == dmin, cidx, _NUM_CODES)
        idx_ref[r:r + _RS, :] = jnp.min(cand, axis=1, keepdims=True)
        ps = jnp.sum(dmin)
        part = ps if part is None else part + ps
    return part


def _fast_body(z_ref, cb_ref, idx_ref, loss_ref, ok_ref, acc_ref, okacc_ref):
    i = pl.program_id(0)
    z = z_ref[...]
    cb = cb_ref[...]
    zsq = jnp.sum(z * z, axis=1, keepdims=True)
    mm2 = lax.dot_general(z + z, cb, (((1,), (1,)), ((), ())),
                          preferred_element_type=jnp.float32)

    stripe = _NUM_CODES // (_N_TOKENS // _BM)
    cs = cb_ref[pl.ds(i * stripe, stripe), :]
    csq_part = jnp.max(jnp.sum(cs * cs, axis=1))
    blk_ok = jnp.logical_and(jnp.min(zsq) >= 129.0,
                             csq_part < _CSQ_BOUND_SAFE).astype(jnp.int32)

    @pl.when(i == 0)
    def _():
        acc_ref[0] = 0.0
        okacc_ref[0] = 1

    okacc_ref[0] = jnp.minimum(okacc_ref[0], blk_ok)
    acc_ref[0] += _sweep(mm2, zsq, None, idx_ref, True)
    loss_ref[0] = acc_ref[0] * _SCALE
    ok_ref[0] = okacc_ref[0]


def _exact_body(z_ref, cb_ref, csq_ref, zsq_ref, idx_ref, loss_ref, acc_ref):
    i = pl.program_id(0)
    z = z_ref[...]
    cb = cb_ref[...]
    csq = csq_ref[...]
    mm2 = lax.dot_general(z + z, cb, (((1,), (1,)), ((), ())),
                          preferred_element_type=jnp.float32)

    @pl.when(i == 0)
    def _():
        acc_ref[0] = 0.0

    acc_ref[0] += _sweep(mm2, zsq_ref[...], csq, idx_ref, False)
    loss_ref[0] = acc_ref[0] * _SCALE


def _fast_call(z, codebook):
    steps = _N_TOKENS // _BM
    return pl.pallas_call(
        _fast_body,
        grid=(steps,),
        in_specs=[
            pl.BlockSpec((_BM, _CODE_DIM), lambda i: (i, 0)),
            pl.BlockSpec((_NUM_CODES, _CODE_DIM), lambda i: (0, 0)),
        ],
        out_specs=[
            pl.BlockSpec((_BM, 1), lambda i: (i, 0)),
            pl.BlockSpec(memory_space=pltpu.SMEM),
            pl.BlockSpec(memory_space=pltpu.SMEM),
        ],
        out_shape=[
            jax.ShapeDtypeStruct((_N_TOKENS, 1), jnp.int32),
            jax.ShapeDtypeStruct((1,), jnp.float32),
            jax.ShapeDtypeStruct((1,), jnp.int32),
        ],
        scratch_shapes=[pltpu.SMEM((1,), jnp.float32),
                        pltpu.SMEM((1,), jnp.int32)],
    )(z, codebook)


def _exact_call(z, codebook, csq_row, zsq):
    steps = _N_TOKENS // _BM
    return pl.pallas_call(
        _exact_body,
        grid=(steps,),
        in_specs=[
            pl.BlockSpec((_BM, _CODE_DIM), lambda i: (i, 0)),
            pl.BlockSpec((_NUM_CODES, _CODE_DIM), lambda i: (0, 0)),
            pl.BlockSpec((1, _NUM_CODES), lambda i: (0, 0)),
            pl.BlockSpec((_BM, 1), lambda i: (i, 0)),
        ],
        out_specs=[
            pl.BlockSpec((_BM, 1), lambda i: (i, 0)),
            pl.BlockSpec(memory_space=pltpu.SMEM),
        ],
        out_shape=[
            jax.ShapeDtypeStruct((_N_TOKENS, 1), jnp.int32),
            jax.ShapeDtypeStruct((1,), jnp.float32),
        ],
        scratch_shapes=[pltpu.SMEM((1,), jnp.float32)],
    )(z, codebook, csq_row, zsq)


_N_WORKERS = 32
_B_PER_W = _N_TOKENS // _N_WORKERS
_CHUNK = 128


def _gather_body(idx_hbm, cb_hbm, out_hbm, idx_v, rows_v, sem):
    wid = lax.axis_index("s") * 2 + lax.axis_index("c")
    for c in range(_B_PER_W // _CHUNK):
        base = wid * _B_PER_W + c * _CHUNK
        pltpu.sync_copy(idx_hbm.at[pl.ds(base, _CHUNK)], idx_v)
        pltpu.async_copy(cb_hbm.at[idx_v], rows_v, sem).wait()
        pltpu.sync_copy(rows_v, out_hbm.at[pl.ds(base, _CHUNK)])


def _gather_rows(indices, codebook):
    mesh = plsc.VectorSubcoreMesh(core_axis_name="c", subcore_axis_name="s")
    gk = functools.partial(
        pl.kernel,
        mesh=mesh,
        out_type=jax.ShapeDtypeStruct((_N_TOKENS, _CODE_DIM), jnp.float32),
        scratch_types=[
            pltpu.VMEM((_CHUNK,), jnp.int32),
            pltpu.VMEM((_CHUNK, _CODE_DIM), jnp.float32),
            pltpu.SemaphoreType.DMA,
        ],
    )(_gather_body)
    return gk(indices, codebook)


def kernel(z, codebook):
    idx_f, loss_f, ok = _fast_call(z, codebook)

    def use_fast():
        return idx_f, loss_f[0]

    def do_exact():
        csq_row = jnp.sum(codebook * codebook, axis=1).reshape(1, _NUM_CODES)
        zsq = jnp.sum(z * z, axis=1, keepdims=True)
        idx_e, loss_e = _exact_call(z, codebook, csq_row, zsq)
        return idx_e, loss_e[0]

    idx, loss = lax.cond(ok[0] > 0, use_fast, do_exact)
    indices = idx.reshape(_N_TOKENS)
    z_q = _gather_rows(indices, codebook)
    return (z_q, indices, loss)
